# baseline (speedup 1.0000x reference)
import functools

import jax
import jax.numpy as jnp
from jax import lax
from jax.experimental import pallas as pl
from jax.experimental.pallas import tpu as pltpu
from jax.experimental.pallas import tpu_sc as plsc

FDIM = 768
N_E = 1024
E_DIM = 256
W = 256
BETA = 0.25
N_TOK = 32 * 576
TILE = 512
N_HALF = N_TOK // 2
HALF_TILES = N_HALF // TILE
AUG = FDIM + 128

NC = 2
NS = 16
NW = NC * NS
B_PER_W = N_HALF // NW
CH = 96
N_CH = B_PER_W // CH


def _ln(x, g, b):
    m = jnp.mean(x, axis=-1, keepdims=True)
    v = jnp.mean((x - m) ** 2, axis=-1, keepdims=True)
    return (x - m) / jnp.sqrt(v + 1e-5) * g + b


def _silu(x):
    return x * jax.nn.sigmoid(x)


def _mm(a, b):
    return jax.lax.dot_general(a, b, (((1,), (0,)), ((), ())),
                               preferred_element_type=jnp.float32)


def _prep_kernel(emb, g0, b0, w1, b1, g1, b1b, w2, b2, g2, b2b, w3, b3,
                 cb_out, dec_out, daug_out, nd2_out):
    e = emb[...]
    n = jnp.sqrt(jnp.sum(e * e, axis=-1, keepdims=True))
    cb = e / jnp.maximum(n, 1e-12)
    cb_out[...] = cb
    x = _ln(cb, g0[...], b0[...])
    x = _silu(_mm(x, w1[...]) + b1[...])
    x = _ln(x, g1[...], b1b[...])
    x = _silu(_mm(x, w2[...]) + b2[...])
    x = _ln(x, g2[...], b2b[...])
    dec = _mm(x, w3[...]) + b3[...]
    dec_out[...] = dec
    nd2 = jnp.sum(dec * dec, axis=1, keepdims=True)
    nd2_out[...] = nd2
    ndc = jnp.maximum(jnp.sqrt(nd2), 1e-6)
    daug_out[...] = jnp.concatenate(
        [dec, jnp.broadcast_to(ndc, (N_E, 128))], axis=1)


def _make_main(half):
    def _main_kernel(z_ref, eg0, eb0, ew1, ebi1, eg1, ebb1, ew2, ebi2,
                     eg2, ebb2, ew3, ebi3, cb_ref, daug_ref,
                     h_out, idx_out, hist_out, psum_out,
                     msum_out, csum_out, xsum_out, z2sum_out,
                     psum_acc, pcomp_acc, m_acc, c_acc, x_acc, z2_acc):
        step = pl.program_id(0)

        @pl.when(step == 0)
        def _init():
            psum_acc[...] = jnp.zeros_like(psum_acc)
            pcomp_acc[...] = jnp.zeros_like(pcomp_acc)
            m_acc[...] = jnp.zeros_like(m_acc)
            c_acc[...] = jnp.zeros_like(c_acc)
            x_acc[...] = jnp.zeros_like(x_acc)
            z2_acc[...] = jnp.zeros_like(z2_acc)
            hist_out[...] = jnp.zeros_like(hist_out)

        z = z_ref[...]
        x = _ln(z, eg0[...], eb0[...])
        x = _silu(_mm(x, ew1[...]) + ebi1[...])
        x = _ln(x, eg1[...], ebb1[...])
        x = _silu(_mm(x, ew2[...]) + ebi2[...])
        x = _ln(x, eg2[...], ebb2[...])
        h = _mm(x, ew3[...]) + ebi3[...]
        hn = jnp.sqrt(jnp.sum(h * h, axis=-1, keepdims=True))
        h = h / jnp.maximum(hn, 1e-12)
        h_out[...] = h

        cb = cb_ref[...]
        hn2 = jnp.sqrt(jnp.sum(h * h, axis=-1, keepdims=True))
        hd = h / jnp.maximum(hn2, 1e-12)
        d = jax.lax.dot_general(hd, cb, (((1,), (1,)), ((), ())),
                                preferred_element_type=jnp.float32)
        m = jnp.max(d, axis=1, keepdims=True)
        iota = jax.lax.broadcasted_iota(jnp.int32, d.shape, 1)
        idx = jnp.min(jnp.where(d == m, iota, N_E), axis=1)
        idx_out[...] = idx[:, None]
        one_hot = (iota == idx[:, None]).astype(jnp.float32)
        e = jnp.exp(d - m)
        s = jnp.sum(e, axis=1, keepdims=True)
        prob = e / s
        x_new = jnp.sum(prob, axis=0)[None, :] - pcomp_acc[...]
        t = psum_acc[...] + x_new
        pcomp_acc[...] = (t - psum_acc[...]) - x_new
        psum_acc[...] = t
        hist_out[...] += jnp.sum(one_hot, axis=0)[None, :]

        p_aug = _mm(one_hot, daug_ref[...])
        zhat = p_aug[:, :FDIM]
        ndi = p_aug[:, FDIM:FDIM + 1]
        z2 = jnp.sum(z * z, axis=-1, keepdims=True)
        zn = jnp.sqrt(z2)
        rdot = jnp.sum(zhat * z, axis=-1, keepdims=True)
        cosr = rdot / (ndi * jnp.maximum(zn, 1e-6))

        m_acc[...] += jnp.sum(m, axis=0, keepdims=True)
        c_acc[...] += jnp.sum(cosr, axis=0, keepdims=True)
        x_acc[...] += jnp.sum(rdot, axis=0, keepdims=True)
        z2_acc[...] += jnp.sum(z2, axis=0, keepdims=True)

        @pl.when(step == HALF_TILES - 1)
        def _finalize():
            psum_out[...] = psum_acc[...]
            msum_out[...] = m_acc[...]
            csum_out[...] = c_acc[...]
            xsum_out[...] = x_acc[...]
            z2sum_out[...] = z2_acc[...]

    return _main_kernel


def _epilogue_kernel(nd2_ref, hist0, hist1, psum0, psum1,
                     msum0, msum1, csum0, csum1, xsum0, xsum1,
                     z20, z21,
                     rec_out, commit_out, kl_out, lb_out, perp_out):
    n_tok = jnp.float32(N_TOK)
    hist = hist0[...] + hist1[...]
    psum = psum0[...] + psum1[...]
    hn2 = jax.lax.dot_general(hist, nd2_ref[...], (((1,), (0,)), ((), ())),
                              preferred_element_type=jnp.float32)
    mse = hn2 - 2.0 * (xsum0[...] + xsum1[...]) + z20[...] + z21[...]
    rec_out[...] = (1.0 - (csum0[...] + csum1[...]) / n_tok
                    + 0.001 * mse / (n_tok * FDIM))
    commit_out[...] = ((1.0 + BETA)
                       * (1.0 - (msum0[...] + msum1[...]) / n_tok))
    e_mean = hist / n_tok
    p = psum / n_tok
    kl_out[...] = jnp.sum(p * (jnp.log(p) - jnp.log(1.0 / N_E)),
                          axis=1, keepdims=True)
    lb_out[...] = jnp.sum(e_mean * p, axis=1, keepdims=True)
    perp_out[...] = jnp.exp(-jnp.sum(e_mean * jnp.log(e_mean + 1e-6),
                                     axis=1, keepdims=True))


def _gather_call(dec_tab, idx3d):
    mesh = plsc.VectorSubcoreMesh(core_axis_name="c", subcore_axis_name="s")

    @functools.partial(
        pl.kernel, mesh=mesh,
        out_type=jax.ShapeDtypeStruct((N_HALF, FDIM), jnp.float32),
        scratch_types=[pltpu.VMEM((N_CH, CH), jnp.int32),
                       pltpu.VMEM((CH, FDIM), jnp.float32),
                       pltpu.SemaphoreType.DMA],
    )
    def k(table_hbm, idx_hbm, out_hbm, idx_v, rows_v, sem):
        wid = lax.axis_index("s") * NC + lax.axis_index("c")
        pltpu.sync_copy(idx_hbm.at[wid], idx_v)
        base = wid * B_PER_W

        def body(j, carry):
            pltpu.async_copy(table_hbm.at[idx_v.at[j]], rows_v, sem).wait()
            pltpu.sync_copy(rows_v, out_hbm.at[pl.ds(base + j * CH, CH)])
            return carry

        lax.fori_loop(0, N_CH, body, 0)

    return k(dec_tab, idx3d)


def _row(v):
    return v.reshape(1, -1)


@jax.jit
def kernel(z, params):
    f32 = jnp.float32
    sds = jax.ShapeDtypeStruct
    z_flat = z.reshape(-1, FDIM)

    prep_args = (params['emb'],
                 _row(params['dec_ln0_g']), _row(params['dec_ln0_b']),
                 params['dec_w1'], _row(params['dec_b1']),
                 _row(params['dec_ln1_g']), _row(params['dec_ln1_b']),
                 params['dec_w2'], _row(params['dec_b2']),
                 _row(params['dec_ln2_g']), _row(params['dec_ln2_b']),
                 params['dec_w3'], _row(params['dec_b3']))
    cb, dec_tab, daug, nd2 = pl.pallas_call(
        _prep_kernel,
        out_shape=(sds((N_E, E_DIM), f32), sds((N_E, FDIM), f32),
                   sds((N_E, AUG), f32), sds((N_E, 1), f32)),
    )(*prep_args)

    enc_args = (_row(params['enc_ln0_g']), _row(params['enc_ln0_b']),
                params['enc_w1'], _row(params['enc_b1']),
                _row(params['enc_ln1_g']), _row(params['enc_ln1_b']),
                params['enc_w2'], _row(params['enc_b2']),
                _row(params['enc_ln2_g']), _row(params['enc_ln2_b']),
                params['enc_w3'], _row(params['enc_b3']))

    full = lambda a: pl.BlockSpec(a.shape, lambda i: (0, 0))
    vec_spec = pl.BlockSpec((1, N_E), lambda i: (0, 0))
    scal_spec = pl.BlockSpec((1, 1), lambda i: (0, 0))
    scal = sds((1, 1), f32)

    def run_half(half, h_prev):
        in_specs = [pl.BlockSpec((TILE, FDIM),
                                 lambda i, _h=half: (i + _h * HALF_TILES, 0))]
        for a in enc_args:
            in_specs.append(full(a))
        in_specs.append(pl.BlockSpec((N_E, E_DIM), lambda i: (0, 0)))
        in_specs.append(pl.BlockSpec((N_E, AUG), lambda i: (0, 0)))
        args = [z_flat, *enc_args, cb, daug]
        kwargs = {}
        if h_prev is not None:
            in_specs.append(pl.BlockSpec(memory_space=pl.ANY))
            args.append(h_prev)
            kwargs = dict(input_output_aliases={len(args) - 1: 0})
        out_shape = (sds((N_TOK, E_DIM), f32),
                     sds((N_HALF, 1), jnp.int32),
                     sds((1, N_E), f32),
                     sds((1, N_E), f32),
                     scal, scal, scal, scal)
        out_specs = (pl.BlockSpec((TILE, E_DIM),
                                  lambda i, _h=half: (i + _h * HALF_TILES, 0)),
                     pl.BlockSpec((TILE, 1), lambda i: (i, 0)),
                     vec_spec, vec_spec,
                     scal_spec, scal_spec, scal_spec, scal_spec)
        body = _make_main(half)
        if h_prev is not None:
            def body(*refs, _inner=_make_main(half)):
                _inner(*refs[:15], *refs[16:])
        return pl.pallas_call(
            body,
            grid=(HALF_TILES,),
            in_specs=in_specs,
            out_specs=out_specs,
            out_shape=out_shape,
            scratch_shapes=[pltpu.VMEM((1, N_E), f32),
                            pltpu.VMEM((1, N_E), f32),
                            pltpu.VMEM((1, 1), f32),
                            pltpu.VMEM((1, 1), f32),
                            pltpu.VMEM((1, 1), f32),
                            pltpu.VMEM((1, 1), f32)],
            compiler_params=pltpu.CompilerParams(
                dimension_semantics=("arbitrary",)),
            **kwargs,
        )(*args)

    h0, idx0, hist0, psum0, ms0, cs0, xs0, z20 = run_half(0, None)
    zhat0 = _gather_call(dec_tab, idx0.reshape(NW, N_CH, CH))
    h, idx1, hist1, psum1, ms1, cs1, xs1, z21 = run_half(1, h0)
    zhat1 = _gather_call(dec_tab, idx1.reshape(NW, N_CH, CH))
    zhat = jnp.concatenate([zhat0, zhat1], axis=0)

    rec, commit, kl, lb, perp = pl.pallas_call(
        _epilogue_kernel,
        out_shape=(scal, scal, scal, scal, scal),
    )(nd2, hist0, hist1, psum0, psum1, ms0, ms1, cs0, cs1, xs0, xs1,
      z20, z21)

    return (zhat, rec[0, 0], commit[0, 0], kl[0, 0], lb[0, 0], cb, h,
            perp[0, 0])


# speedup vs baseline: 1.6773x; 1.6773x over previous
import jax
import jax.numpy as jnp
from jax.experimental import pallas as pl
from jax.experimental.pallas import tpu as pltpu

FDIM = 768
N_E = 1024
E_DIM = 256
W = 256
BETA = 0.25
N_TOK = 32 * 576
TILE = 512
N_TILES = N_TOK // TILE


def _ln(x, g, b):
    m = jnp.mean(x, axis=-1, keepdims=True)
    v = jnp.mean((x - m) ** 2, axis=-1, keepdims=True)
    return (x - m) / jnp.sqrt(v + 1e-5) * g + b


def _silu(x):
    return x * jax.nn.sigmoid(x)


def _mm(a, b):
    return jax.lax.dot_general(a, b, (((1,), (0,)), ((), ())),
                               preferred_element_type=jnp.float32)


def _prep_kernel(emb, g0, b0, w1, b1, g1, b1b, w2, b2, g2, b2b, w3, b3,
                 cb_out, dec_out):
    e = emb[...]
    n = jnp.sqrt(jnp.sum(e * e, axis=-1, keepdims=True))
    cb = e / jnp.maximum(n, 1e-12)
    cb_out[...] = cb
    x = _ln(cb, g0[...], b0[...])
    x = _silu(_mm(x, w1[...]) + b1[...])
    x = _ln(x, g1[...], b1b[...])
    x = _silu(_mm(x, w2[...]) + b2[...])
    x = _ln(x, g2[...], b2b[...])
    dec_out[...] = _mm(x, w3[...]) + b3[...]


def _main_kernel(z_ref, eg0, eb0, ew1, ebi1, eg1, ebb1, ew2, ebi2, eg2, ebb2,
                 ew3, ebi3, cb_ref, d_ref,
                 h_out, zhat_out, rec_out, commit_out, kl_out, lb_out, perp_out,
                 psum_acc, pcomp_acc, hist_acc, m_acc, c_acc, sq_acc):
    step = pl.program_id(0)

    @pl.when(step == 0)
    def _init():
        psum_acc[...] = jnp.zeros_like(psum_acc)
        pcomp_acc[...] = jnp.zeros_like(pcomp_acc)
        hist_acc[...] = jnp.zeros_like(hist_acc)
        m_acc[...] = jnp.zeros_like(m_acc)
        c_acc[...] = jnp.zeros_like(c_acc)
        sq_acc[...] = jnp.zeros_like(sq_acc)

    z = z_ref[...]
    x = _ln(z, eg0[...], eb0[...])
    x = _silu(_mm(x, ew1[...]) + ebi1[...])
    x = _ln(x, eg1[...], ebb1[...])
    x = _silu(_mm(x, ew2[...]) + ebi2[...])
    x = _ln(x, eg2[...], ebb2[...])
    h = _mm(x, ew3[...]) + ebi3[...]
    hn = jnp.sqrt(jnp.sum(h * h, axis=-1, keepdims=True))
    h = h / jnp.maximum(hn, 1e-12)
    h_out[...] = h

    cb = cb_ref[...]
    hn2 = jnp.sqrt(jnp.sum(h * h, axis=-1, keepdims=True))
    hd = h / jnp.maximum(hn2, 1e-12)
    d = jax.lax.dot_general(hd, cb, (((1,), (1,)), ((), ())),
                            preferred_element_type=jnp.float32)
    m = jnp.max(d, axis=1, keepdims=True)
    iota = jax.lax.broadcasted_iota(jnp.int32, d.shape, 1)
    idx = jnp.min(jnp.where(d == m, iota, N_E), axis=1)
    one_hot = (iota == idx[:, None]).astype(jnp.float32)
    e = jnp.exp(d - m)
    s = jnp.sum(e, axis=1, keepdims=True)
    prob = e / s
    x_new = jnp.sum(prob, axis=0)[None, :] - pcomp_acc[...]
    t = psum_acc[...] + x_new
    pcomp_acc[...] = (t - psum_acc[...]) - x_new
    psum_acc[...] = t
    hist_acc[...] += jnp.sum(one_hot, axis=0)[None, :]

    zhat = _mm(one_hot, d_ref[...])
    zhat_out[...] = zhat

    zn = jnp.sqrt(jnp.sum(z * z, axis=-1, keepdims=True))
    nzh = jnp.sqrt(jnp.sum(zhat * zhat, axis=-1, keepdims=True))
    dot = jnp.sum(zhat * z, axis=-1, keepdims=True)
    cosr = dot / (jnp.maximum(nzh, 1e-6) * jnp.maximum(zn, 1e-6))
    diff = zhat - z
    m_acc[...] += jnp.sum(m, axis=0, keepdims=True)
    c_acc[...] += jnp.sum(cosr, axis=0, keepdims=True)
    sq_acc[...] += jnp.sum(jnp.sum(diff * diff, axis=1, keepdims=True),
                           axis=0, keepdims=True)

    @pl.when(step == N_TILES - 1)
    def _finalize():
        n_tok = jnp.float32(N_TOK)
        e_mean = hist_acc[...] / n_tok
        p = psum_acc[...] / n_tok
        kl = jnp.sum(p * (jnp.log(p) - jnp.log(1.0 / N_E)),
                     axis=1, keepdims=True)
        lb = jnp.sum(e_mean * p, axis=1, keepdims=True)
        perp = jnp.exp(-jnp.sum(e_mean * jnp.log(e_mean + 1e-6),
                                axis=1, keepdims=True))
        commit = (1.0 + BETA) * (1.0 - m_acc[...] / n_tok)
        rec = (1.0 - c_acc[...] / n_tok
               + 0.001 * sq_acc[...] / (n_tok * FDIM))
        kl_out[...] = kl
        lb_out[...] = lb
        perp_out[...] = perp
        commit_out[...] = commit
        rec_out[...] = rec


def _row(v):
    return v.reshape(1, -1)


@jax.jit
def kernel(z, params):
    f32 = jnp.float32
    sds = jax.ShapeDtypeStruct
    z_flat = z.reshape(-1, FDIM)

    prep_args = (params['emb'],
                 _row(params['dec_ln0_g']), _row(params['dec_ln0_b']),
                 params['dec_w1'], _row(params['dec_b1']),
                 _row(params['dec_ln1_g']), _row(params['dec_ln1_b']),
                 params['dec_w2'], _row(params['dec_b2']),
                 _row(params['dec_ln2_g']), _row(params['dec_ln2_b']),
                 params['dec_w3'], _row(params['dec_b3']))
    cb, dec_tab = pl.pallas_call(
        _prep_kernel,
        out_shape=(sds((N_E, E_DIM), f32), sds((N_E, FDIM), f32)),
    )(*prep_args)

    enc_args = (_row(params['enc_ln0_g']), _row(params['enc_ln0_b']),
                params['enc_w1'], _row(params['enc_b1']),
                _row(params['enc_ln1_g']), _row(params['enc_ln1_b']),
                params['enc_w2'], _row(params['enc_b2']),
                _row(params['enc_ln2_g']), _row(params['enc_ln2_b']),
                params['enc_w3'], _row(params['enc_b3']))

    full = lambda a: pl.BlockSpec(a.shape, lambda i: (0, 0))
    in_specs = [pl.BlockSpec((TILE, FDIM), lambda i: (i, 0))]
    for a in enc_args:
        in_specs.append(full(a))
    in_specs.append(pl.BlockSpec((N_E, E_DIM), lambda i: (0, 0)))
    in_specs.append(pl.BlockSpec((N_E, FDIM), lambda i: (0, 0)))

    scal = sds((1, 1), f32)
    scal_spec = pl.BlockSpec((1, 1), lambda i: (0, 0))
    out_shape = (sds((N_TOK, E_DIM), f32),
                 sds((N_TOK, FDIM), f32),
                 scal, scal, scal, scal, scal)
    out_specs = (pl.BlockSpec((TILE, E_DIM), lambda i: (i, 0)),
                 pl.BlockSpec((TILE, FDIM), lambda i: (i, 0)),
                 scal_spec, scal_spec, scal_spec, scal_spec, scal_spec)

    h, zhat, rec, commit, kl, lb, perp = pl.pallas_call(
        _main_kernel,
        grid=(N_TILES,),
        in_specs=in_specs,
        out_specs=out_specs,
        out_shape=out_shape,
        scratch_shapes=[pltpu.VMEM((1, N_E), f32),
                        pltpu.VMEM((1, N_E), f32),
                        pltpu.VMEM((1, N_E), f32),
                        pltpu.VMEM((1, 1), f32),
                        pltpu.VMEM((1, 1), f32),
                        pltpu.VMEM((1, 1), f32)],
        compiler_params=pltpu.CompilerParams(
            dimension_semantics=("arbitrary",)),
    )(z_flat, *enc_args, cb, dec_tab)

    return (zhat, rec[0, 0], commit[0, 0], kl[0, 0], lb[0, 0], cb, h,
            perp[0, 0])
